# baseline (speedup 1.0000x reference)
import math

import jax
import jax.numpy as jnp
from jax import lax
from jax.experimental import pallas as pl
from jax.experimental.pallas import tpu as pltpu
from jax.experimental.pallas import tpu_sc as plsc

_EMBED = 128
_CHUNK = 128
_SC_ROWS = 8192
_TC_BLK = 1024


def _sc_gather(timestep_sc, time_embs):
    rows = timestep_sc.shape[0]
    mesh = plsc.VectorSubcoreMesh(core_axis_name="c", subcore_axis_name="s")
    nw = mesh.num_cores * mesh.num_subcores
    n_chunks = rows // (nw * _CHUNK)
    idx2d = timestep_sc.reshape((nw * n_chunks, _CHUNK))

    @pl.kernel(
        out_type=jax.ShapeDtypeStruct((rows, _EMBED), time_embs.dtype),
        mesh=mesh,
        scratch_types=[
            pltpu.VMEM((n_chunks, _CHUNK), jnp.int32),
            pltpu.VMEM((n_chunks, _CHUNK, _EMBED), jnp.float32),
            pltpu.SemaphoreType.DMA((n_chunks,)),
            pltpu.SemaphoreType.DMA((n_chunks,)),
        ],
    )
    def gather_kernel(table_hbm, idx_hbm, out_hbm, idx_v, buf_v, gsem, wsem):
        wid = lax.axis_index("s") * mesh.num_cores + lax.axis_index("c")
        pltpu.sync_copy(idx_hbm.at[pl.ds(wid * n_chunks, n_chunks)], idx_v)

        gathers = []
        for j in range(n_chunks):
            gathers.append(pltpu.async_copy(
                table_hbm.at[idx_v.at[j]], buf_v.at[j], gsem.at[j]))
        writes = []
        for j in range(n_chunks):
            gathers[j].wait()
            dst = out_hbm.at[pl.ds((wid * n_chunks + j) * _CHUNK, _CHUNK)]
            writes.append(pltpu.async_copy(buf_v.at[j], dst, wsem.at[j]))
        for w in writes:
            w.wait()

    return gather_kernel(time_embs, idx2d)


def _tc_sincos_kernel(t_ref, sf_ref, off_ref, out_ref):
    phase = t_ref[...] * sf_ref[...] + off_ref[...]
    out_ref[...] = jnp.sin(phase)


def kernel(timestep, time_embs):
    batch = timestep.shape[0]
    tc_rows = batch - _SC_ROWS
    n_tc_blocks = tc_rows // _TC_BLK
    sc_block0 = _SC_ROWS // _TC_BLK

    scales = jnp.exp(jnp.arange(0, _EMBED, 2, dtype=jnp.float32)
                     * (-math.log(10000.0) / _EMBED))
    sf = jnp.repeat(scales, 2).reshape(1, _EMBED)
    off = jnp.tile(jnp.array([0.0, math.pi / 2], jnp.float32),
                   _EMBED // 2).reshape(1, _EMBED)

    t_col = timestep[_SC_ROWS:].astype(jnp.float32).reshape(tc_rows, 1)

    tc_full = pl.pallas_call(
        _tc_sincos_kernel,
        grid=(n_tc_blocks,),
        in_specs=[
            pl.BlockSpec((_TC_BLK, 1), lambda i: (i, 0)),
            pl.BlockSpec((1, _EMBED), lambda i: (0, 0)),
            pl.BlockSpec((1, _EMBED), lambda i: (0, 0)),
        ],
        out_specs=pl.BlockSpec((_TC_BLK, _EMBED),
                               lambda i: (i + sc_block0, 0)),
        out_shape=jax.ShapeDtypeStruct((batch, _EMBED), jnp.float32),
    )(t_col, sf, off)

    sc_out = _sc_gather(timestep[:_SC_ROWS], time_embs)
    return lax.dynamic_update_slice(tc_full, sc_out, (0, 0))


# speedup vs baseline: 1.1361x; 1.1361x over previous
import math

import jax
import jax.numpy as jnp
from jax import lax
from jax.experimental import pallas as pl
from jax.experimental.pallas import tpu as pltpu
from jax.experimental.pallas import tpu_sc as plsc

_EMBED = 128
_CHUNK = 128
_SC_ROWS = 8192
_TC_BLK = 1024


def _sc_gather(timestep_sc, time_embs):
    rows = timestep_sc.shape[0]
    mesh = plsc.VectorSubcoreMesh(core_axis_name="c", subcore_axis_name="s")
    nw = mesh.num_cores * mesh.num_subcores
    n_chunks = rows // (nw * _CHUNK)
    idx2d = timestep_sc.reshape((nw * n_chunks, _CHUNK))

    @pl.kernel(
        out_type=jax.ShapeDtypeStruct((rows, _EMBED), time_embs.dtype),
        mesh=mesh,
        scratch_types=[
            pltpu.VMEM((n_chunks, _CHUNK), jnp.int32),
            pltpu.VMEM((n_chunks, _CHUNK, _EMBED), jnp.float32),
            pltpu.SemaphoreType.DMA((n_chunks,)),
            pltpu.SemaphoreType.DMA((n_chunks,)),
        ],
    )
    def gather_kernel(table_hbm, idx_hbm, out_hbm, idx_v, buf_v, gsem, wsem):
        wid = lax.axis_index("s") * mesh.num_cores + lax.axis_index("c")
        pltpu.sync_copy(idx_hbm.at[pl.ds(wid * n_chunks, n_chunks)], idx_v)

        gathers = []
        for j in range(n_chunks):
            gathers.append(pltpu.async_copy(
                table_hbm.at[idx_v.at[j]], buf_v.at[j], gsem.at[j]))
        writes = []
        for j in range(n_chunks):
            gathers[j].wait()
            dst = out_hbm.at[pl.ds((wid * n_chunks + j) * _CHUNK, _CHUNK)]
            writes.append(pltpu.async_copy(buf_v.at[j], dst, wsem.at[j]))
        for w in writes:
            w.wait()

    return gather_kernel(time_embs, idx2d)


def _tc_sincos_kernel(t_ref, sf_ref, off_ref, out_ref):
    phase = t_ref[...] * sf_ref[...] + off_ref[...]
    out_ref[...] = jnp.sin(phase)


def kernel(timestep, time_embs):
    batch = timestep.shape[0]
    scales = jnp.exp(jnp.arange(0, _EMBED, 2, dtype=jnp.float32)
                     * (-math.log(10000.0) / _EMBED))
    sf = jnp.repeat(scales, 2).reshape(1, _EMBED)
    off = jnp.tile(jnp.array([0.0, math.pi / 2], jnp.float32),
                   _EMBED // 2).reshape(1, _EMBED)
    t_col = timestep.astype(jnp.float32).reshape(batch, 1)
    return pl.pallas_call(
        _tc_sincos_kernel,
        grid=(batch // _TC_BLK,),
        in_specs=[
            pl.BlockSpec((_TC_BLK, 1), lambda i: (i, 0)),
            pl.BlockSpec((1, _EMBED), lambda i: (0, 0)),
            pl.BlockSpec((1, _EMBED), lambda i: (0, 0)),
        ],
        out_specs=pl.BlockSpec((_TC_BLK, _EMBED), lambda i: (i, 0)),
        out_shape=jax.ShapeDtypeStruct((batch, _EMBED), jnp.float32),
    )(t_col, sf, off)


def _unused_kernel(timestep, time_embs):
    batch = timestep.shape[0]
    tc_rows = batch - _SC_ROWS
    n_tc_blocks = tc_rows // _TC_BLK
    sc_block0 = _SC_ROWS // _TC_BLK

    scales = jnp.exp(jnp.arange(0, _EMBED, 2, dtype=jnp.float32)
                     * (-math.log(10000.0) / _EMBED))
    sf = jnp.repeat(scales, 2).reshape(1, _EMBED)
    off = jnp.tile(jnp.array([0.0, math.pi / 2], jnp.float32),
                   _EMBED // 2).reshape(1, _EMBED)

    t_col = timestep[_SC_ROWS:].astype(jnp.float32).reshape(tc_rows, 1)

    tc_full = pl.pallas_call(
        _tc_sincos_kernel,
        grid=(n_tc_blocks,),
        in_specs=[
            pl.BlockSpec((_TC_BLK, 1), lambda i: (i, 0)),
            pl.BlockSpec((1, _EMBED), lambda i: (0, 0)),
            pl.BlockSpec((1, _EMBED), lambda i: (0, 0)),
        ],
        out_specs=pl.BlockSpec((_TC_BLK, _EMBED),
                               lambda i: (i + sc_block0, 0)),
        out_shape=jax.ShapeDtypeStruct((batch, _EMBED), jnp.float32),
    )(t_col, sf, off)

    sc_out = _sc_gather(timestep[:_SC_ROWS], time_embs)
    return lax.dynamic_update_slice(tc_full, sc_out, (0, 0))


# speedup vs baseline: 1.8682x; 1.6443x over previous
import math

import jax
import jax.numpy as jnp
from jax import lax
from jax.experimental import pallas as pl
from jax.experimental.pallas import tpu as pltpu
from jax.experimental.pallas import tpu_sc as plsc

_EMBED = 128
_CHUNK = 128
_SC_ROWS = 8192
_TC_BLK = 1024


def _sc_gather(timestep_sc, time_embs):
    rows = timestep_sc.shape[0]
    mesh = plsc.VectorSubcoreMesh(core_axis_name="c", subcore_axis_name="s")
    nw = mesh.num_cores * mesh.num_subcores
    n_chunks = rows // (nw * _CHUNK)
    idx2d = timestep_sc.reshape((nw * n_chunks, _CHUNK))

    @pl.kernel(
        out_type=jax.ShapeDtypeStruct((rows, _EMBED), time_embs.dtype),
        mesh=mesh,
        scratch_types=[
            pltpu.VMEM((n_chunks, _CHUNK), jnp.int32),
            pltpu.VMEM((n_chunks, _CHUNK, _EMBED), jnp.float32),
            pltpu.SemaphoreType.DMA((n_chunks,)),
            pltpu.SemaphoreType.DMA((n_chunks,)),
        ],
    )
    def gather_kernel(table_hbm, idx_hbm, out_hbm, idx_v, buf_v, gsem, wsem):
        wid = lax.axis_index("s") * mesh.num_cores + lax.axis_index("c")
        pltpu.sync_copy(idx_hbm.at[pl.ds(wid * n_chunks, n_chunks)], idx_v)

        gathers = []
        for j in range(n_chunks):
            gathers.append(pltpu.async_copy(
                table_hbm.at[idx_v.at[j]], buf_v.at[j], gsem.at[j]))
        writes = []
        for j in range(n_chunks):
            gathers[j].wait()
            dst = out_hbm.at[pl.ds((wid * n_chunks + j) * _CHUNK, _CHUNK)]
            writes.append(pltpu.async_copy(buf_v.at[j], dst, wsem.at[j]))
        for w in writes:
            w.wait()

    return gather_kernel(time_embs, idx2d)


def _tc_sincos_kernel(t_ref, sf_ref, off_ref, out_ref):
    r = t_ref[...] * sf_ref[...] + off_ref[...]
    n = jnp.round(r)
    f = r - n
    half = 0.5 * n
    parity = half - jnp.floor(half)
    sign = 1.0 - 4.0 * parity
    y = (f * math.pi) * sign
    z = y * y
    p = ((((2.7557319e-06 * z - 1.9841270e-04) * z + 8.3333333e-03) * z
          - 1.6666667e-01) * z + 1.0) * y
    out_ref[...] = p


def kernel(timestep, time_embs):
    batch = timestep.shape[0]
    scales = jnp.exp(jnp.arange(0, _EMBED, 2, dtype=jnp.float32)
                     * (-math.log(10000.0) / _EMBED))
    sf = (jnp.repeat(scales, 2) / math.pi).reshape(1, _EMBED)
    off = jnp.tile(jnp.array([0.0, 0.5], jnp.float32),
                   _EMBED // 2).reshape(1, _EMBED)
    t_col = timestep.astype(jnp.float32).reshape(batch, 1)
    return pl.pallas_call(
        _tc_sincos_kernel,
        grid=(batch // _TC_BLK,),
        in_specs=[
            pl.BlockSpec((_TC_BLK, 1), lambda i: (i, 0)),
            pl.BlockSpec((1, _EMBED), lambda i: (0, 0)),
            pl.BlockSpec((1, _EMBED), lambda i: (0, 0)),
        ],
        out_specs=pl.BlockSpec((_TC_BLK, _EMBED), lambda i: (i, 0)),
        out_shape=jax.ShapeDtypeStruct((batch, _EMBED), jnp.float32),
    )(t_col, sf, off)


def _unused_kernel(timestep, time_embs):
    batch = timestep.shape[0]
    tc_rows = batch - _SC_ROWS
    n_tc_blocks = tc_rows // _TC_BLK
    sc_block0 = _SC_ROWS // _TC_BLK

    scales = jnp.exp(jnp.arange(0, _EMBED, 2, dtype=jnp.float32)
                     * (-math.log(10000.0) / _EMBED))
    sf = jnp.repeat(scales, 2).reshape(1, _EMBED)
    off = jnp.tile(jnp.array([0.0, math.pi / 2], jnp.float32),
                   _EMBED // 2).reshape(1, _EMBED)

    t_col = timestep[_SC_ROWS:].astype(jnp.float32).reshape(tc_rows, 1)

    tc_full = pl.pallas_call(
        _tc_sincos_kernel,
        grid=(n_tc_blocks,),
        in_specs=[
            pl.BlockSpec((_TC_BLK, 1), lambda i: (i, 0)),
            pl.BlockSpec((1, _EMBED), lambda i: (0, 0)),
            pl.BlockSpec((1, _EMBED), lambda i: (0, 0)),
        ],
        out_specs=pl.BlockSpec((_TC_BLK, _EMBED),
                               lambda i: (i + sc_block0, 0)),
        out_shape=jax.ShapeDtypeStruct((batch, _EMBED), jnp.float32),
    )(t_col, sf, off)

    sc_out = _sc_gather(timestep[:_SC_ROWS], time_embs)
    return lax.dynamic_update_slice(tc_full, sc_out, (0, 0))
